# baseline (speedup 1.0000x reference)
import functools
import math

import jax
import jax.numpy as jnp
from jax import lax
from jax.experimental import pallas as pl
from jax.experimental.pallas import tpu as pltpu

_COARSE_K = 20
_FINE_K = 10
_SEG = 128
_BLK_R = 1024
_BLK_N = 2048
_BIG = 2 ** 30


def _coarse_body(x_ref, w_ref, wenc_ref, logits_ref, sel_ref, query_ref,
                 m_ref, carry_ref, *, nk, ck, g_real):
    i = pl.program_id(0)
    j = pl.program_id(1)
    nj = pl.num_programs(1)
    blk_r, blk_n = logits_ref.shape
    gper = blk_n // _SEG
    per_chunk = 128 // gper

    acc = jnp.dot(x_ref[...], w_ref[...], preferred_element_type=jnp.float32)
    cols = j * blk_n + lax.broadcasted_iota(jnp.int32, (blk_r, blk_n), 1)
    acc = jnp.where(cols < nk, acc, -jnp.inf)
    logits_ref[...] = acc

    @pl.when(j == 0)
    def _():
        query_ref[...] = jnp.dot(x_ref[...], wenc_ref[...],
                                 preferred_element_type=jnp.float32)

    pieces = [jnp.max(acc[:, g * _SEG:(g + 1) * _SEG], axis=1, keepdims=True)
              for g in range(gper)]
    tile_m = jnp.concatenate(pieces, axis=1)
    q = j % per_chunk

    @pl.when(q == 0)
    def _():
        carry_ref[...] = jnp.full(carry_ref.shape, -jnp.inf, jnp.float32)

    for qq in range(per_chunk):
        @pl.when(q == qq)
        def _(qq=qq):
            carry_ref[:, qq * gper:(qq + 1) * gper] = tile_m

    @pl.when((q == per_chunk - 1) | (j == nj - 1))
    def _():
        m_ref[:, pl.ds((j // per_chunk) * 128, 128)] = carry_ref[...]

    @pl.when(j == nj - 1)
    def _():
        g_tot = m_ref.shape[1]
        gi0 = lax.broadcasted_iota(jnp.int32, (blk_r, g_tot), 1)
        giota = gi0
        picks = []
        Mw = jnp.where(gi0 < g_real, m_ref[...], -jnp.inf)
        for _ in range(ck):
            m = jnp.max(Mw, axis=1, keepdims=True)
            pick = jnp.min(jnp.where(Mw == m, giota, _BIG),
                           axis=1, keepdims=True)
            picks.append(pick)
            Mw = jnp.where(giota == pick, -jnp.inf, Mw)
        gids = jnp.concatenate(picks, axis=1)
        outs = []
        for _ in range(ck):
            mn = jnp.min(gids, axis=1, keepdims=True)
            outs.append(mn)
            gids = jnp.where(gids == mn, _BIG, gids)
        gsorted = jnp.concatenate(outs, axis=1)
        rows = i * blk_r + lax.broadcasted_iota(jnp.int32, (blk_r, ck), 0)
        flat = rows * g_real + gsorted
        pad = jnp.zeros((blk_r, sel_ref.shape[1] - ck), jnp.int32)
        sel_ref[...] = jnp.concatenate([flat, pad], axis=1)


def _coarse_call(x2, wb, wenc, nk):
    n_rows, d = x2.shape
    blk_r = min(_BLK_R, n_rows)
    nj = (nk + _BLK_N - 1) // _BLK_N
    n_pad = nj * _BLK_N
    g_tot = n_pad // _SEG
    grid = (n_rows // blk_r, nj)
    g_pad = ((g_tot + 127) // 128) * 128
    body = functools.partial(_coarse_body, nk=nk, ck=_COARSE_K, g_real=g_tot)
    return pl.pallas_call(
        body,
        grid=grid,
        in_specs=[
            pl.BlockSpec((blk_r, d), lambda i, j: (i, 0)),
            pl.BlockSpec((d, _BLK_N), lambda i, j: (0, j)),
            pl.BlockSpec((d, 128), lambda i, j: (0, 0)),
        ],
        out_specs=[
            pl.BlockSpec((blk_r, _BLK_N), lambda i, j: (i, j)),
            pl.BlockSpec((blk_r, 128), lambda i, j: (i, 0)),
            pl.BlockSpec((blk_r, 128), lambda i, j: (i, 0)),
        ],
        out_shape=[
            jax.ShapeDtypeStruct((n_rows, n_pad), jnp.float32),
            jax.ShapeDtypeStruct((n_rows, 128), jnp.int32),
            jax.ShapeDtypeStruct((n_rows, 128), jnp.float32),
        ],
        scratch_shapes=[pltpu.VMEM((blk_r, g_pad), jnp.float32),
                        pltpu.VMEM((blk_r, 128), jnp.float32)],
        compiler_params=pltpu.CompilerParams(
            dimension_semantics=("parallel", "arbitrary")),
    )(x2, wb, wenc)


_SC_NC = 2
_SC_NS = 16


def _sc_gather(table, idx, chunk):
    try:
        from jax.experimental.pallas import tpu_sc as plsc
    except ImportError:
        return jnp.take(table, idx, axis=0)
    n = idx.shape[0]
    drow = table.shape[1]
    nw = _SC_NC * _SC_NS
    b_per_w = n // nw
    assert n % nw == 0 and b_per_w % chunk == 0
    nch = b_per_w // chunk
    mesh = plsc.VectorSubcoreMesh(core_axis_name="c", subcore_axis_name="s")

    @functools.partial(
        pl.kernel,
        out_type=jax.ShapeDtypeStruct((n, drow), table.dtype),
        mesh=mesh,
        scratch_types=[
            pltpu.VMEM((chunk,), jnp.int32),
            pltpu.VMEM((chunk, drow), table.dtype),
            pltpu.SemaphoreType.DMA,
        ],
    )
    def k(table_hbm, idx_hbm, out_hbm, idx_v, rows_v, sem):
        wid = lax.axis_index("s") * _SC_NC + lax.axis_index("c")
        base = wid * b_per_w

        def body(c, carry):
            off = base + c * chunk
            pltpu.sync_copy(idx_hbm.at[pl.ds(off, chunk)], idx_v)
            pltpu.async_copy(table_hbm.at[idx_v], rows_v, sem).wait()
            pltpu.sync_copy(rows_v, out_hbm.at[pl.ds(off, chunk)])
            return carry

        lax.fori_loop(0, nch, body, 0)

    return k(table, idx)


_BLK_R2 = 64


def _refine_body(vals_ref, sel_ref, cand_ref, score_ref, *, ck, g_real):
    blk_r, width = vals_ref.shape
    vals = vals_ref[...]
    piota = lax.broadcasted_iota(jnp.int32, (blk_r, width), 1)
    gid = sel_ref[:, :ck] % g_real
    kiota = lax.broadcasted_iota(jnp.int32, (blk_r, ck), 1)
    mcols, gcols = [], []
    for _ in range(ck):
        m = jnp.max(vals, axis=1, keepdims=True)
        p = jnp.min(jnp.where(vals == m, piota, _BIG), axis=1, keepdims=True)
        vals = jnp.where(piota == p, -jnp.inf, vals)
        gid_p = jnp.min(jnp.where(kiota == p // _SEG, gid, _BIG),
                        axis=1, keepdims=True)
        g = gid_p * _SEG + p % _SEG
        mcols.append(m)
        gcols.append(g)
    pad_i = jnp.zeros((blk_r, cand_ref.shape[1] - ck), jnp.int32)
    pad_f = jnp.zeros((blk_r, score_ref.shape[1] - ck), jnp.float32)
    cand_ref[...] = jnp.concatenate(gcols + [pad_i], axis=1)
    score_ref[...] = jnp.concatenate(mcols + [pad_f], axis=1)


def _refine_call(vals, sel, g_real):
    n_rows, width = vals.shape
    blk_r = min(_BLK_R2, n_rows)
    grid = (n_rows // blk_r,)
    body = functools.partial(_refine_body, ck=_COARSE_K, g_real=g_real)
    return pl.pallas_call(
        body,
        grid=grid,
        in_specs=[
            pl.BlockSpec((blk_r, width), lambda i: (i, 0)),
            pl.BlockSpec((blk_r, 128), lambda i: (i, 0)),
        ],
        out_specs=[
            pl.BlockSpec((blk_r, 128), lambda i: (i, 0)),
            pl.BlockSpec((blk_r, 128), lambda i: (i, 0)),
        ],
        out_shape=[
            jax.ShapeDtypeStruct((n_rows, 128), jnp.int32),
            jax.ShapeDtypeStruct((n_rows, 128), jnp.float32),
        ],
        compiler_params=pltpu.CompilerParams(
            dimension_semantics=("parallel",)),
    )(vals, sel)


def _fine_body(q_ref, kg_ref, cand_ref, fgi_ref, fw_ref, *, ck, fk, kr):
    blk_r = q_ref.shape[0]
    qf = q_ref[...].astype(jnp.bfloat16).astype(jnp.float32)
    scols = []
    for c in range(ck):
        kc = kg_ref[:, c * kr:(c + 1) * kr].astype(jnp.bfloat16).astype(jnp.float32)
        s = jnp.sum(qf * kc, axis=1, keepdims=True) / math.sqrt(kr)
        scols.append(s)
    neg = jnp.full((blk_r, 128 - ck), -jnp.inf, jnp.float32)
    fs = jnp.concatenate(scols + [neg], axis=1)
    iota = lax.broadcasted_iota(jnp.int32, (blk_r, 128), 1)
    cand = cand_ref[...]
    tcols, icols = [], []
    for _ in range(fk):
        m = jnp.max(fs, axis=1, keepdims=True)
        p = jnp.min(jnp.where(fs == m, iota, _BIG), axis=1, keepdims=True)
        g = jnp.min(jnp.where(iota == p, cand, _BIG), axis=1, keepdims=True)
        tcols.append(m)
        icols.append(g)
        fs = jnp.where(iota == p, -jnp.inf, fs)
    fts = jnp.concatenate(tcols, axis=1)
    mx = fts[:, 0:1]
    e = jnp.exp(fts - mx)
    w = e / jnp.sum(e, axis=1, keepdims=True)
    pad_i = jnp.zeros((blk_r, fgi_ref.shape[1] - fk), jnp.int32)
    pad_f = jnp.zeros((blk_r, fw_ref.shape[1] - fk), jnp.float32)
    fgi_ref[...] = jnp.concatenate(icols + [pad_i], axis=1)
    fw_ref[...] = jnp.concatenate([w, pad_f], axis=1)


def _fine_call(query, kg, cand, kr):
    n_rows = query.shape[0]
    blk_r = min(_BLK_R2, n_rows)
    grid = (n_rows // blk_r,)
    width = kg.shape[1]
    body = functools.partial(_fine_body, ck=_COARSE_K, fk=_FINE_K, kr=kr)
    return pl.pallas_call(
        body,
        grid=grid,
        in_specs=[
            pl.BlockSpec((blk_r, query.shape[1]), lambda i: (i, 0)),
            pl.BlockSpec((blk_r, width), lambda i: (i, 0)),
            pl.BlockSpec((blk_r, 128), lambda i: (i, 0)),
        ],
        out_specs=[
            pl.BlockSpec((blk_r, 128), lambda i: (i, 0)),
            pl.BlockSpec((blk_r, 128), lambda i: (i, 0)),
        ],
        out_shape=[
            jax.ShapeDtypeStruct((n_rows, 128), jnp.int32),
            jax.ShapeDtypeStruct((n_rows, 128), jnp.float32),
        ],
        compiler_params=pltpu.CompilerParams(
            dimension_semantics=("parallel",)),
    )(query, kg, cand)


def _wsum_body(vg_ref, fw_ref, out_ref, *, fk, d):
    acc = vg_ref[:, 0:d] * fw_ref[:, 0:1]
    for j in range(1, fk):
        acc = acc + vg_ref[:, j * d:(j + 1) * d] * fw_ref[:, j:j + 1]
    out_ref[...] = acc


def _wsum_call(vg, fw, fk, d):
    n_rows = vg.shape[0]
    blk_r = min(_BLK_R2, n_rows)
    grid = (n_rows // blk_r,)
    body = functools.partial(_wsum_body, fk=fk, d=d)
    return pl.pallas_call(
        body,
        grid=grid,
        in_specs=[
            pl.BlockSpec((blk_r, vg.shape[1]), lambda i: (i, 0)),
            pl.BlockSpec((blk_r, 128), lambda i: (i, 0)),
        ],
        out_specs=pl.BlockSpec((blk_r, d), lambda i: (i, 0)),
        out_shape=jax.ShapeDtypeStruct((n_rows, d), jnp.float32),
        compiler_params=pltpu.CompilerParams(
            dimension_semantics=("parallel",)),
    )(vg, fw)


def kernel(x, W_router, W_enc, K_all, V_all):
    B, S, D = x.shape
    NK = W_router.shape[1]
    KR = K_all.shape[1]
    n_rows = B * S

    x2 = x.reshape(n_rows, D).astype(jnp.bfloat16)
    wb = W_router.astype(jnp.bfloat16)
    wenc = W_enc.astype(jnp.bfloat16)

    logits, sel, query = _coarse_call(x2, wb, wenc, NK)
    g_tot = logits.shape[1] // _SEG

    segs = logits.reshape(n_rows * g_tot, _SEG)
    selc = sel[:, :_COARSE_K].reshape(-1)
    gath = _sc_gather(segs, selc, 128)
    vals = gath.reshape(n_rows, _COARSE_K * _SEG)

    cand_pad, score_pad = _refine_call(vals, sel, g_tot)
    candidate_idx = cand_pad[:, :_COARSE_K]
    coarse_scores = score_pad[:, :_COARSE_K]

    kg = _sc_gather(K_all, candidate_idx.reshape(-1), 128)
    kg = kg.reshape(n_rows, _COARSE_K * KR)
    fgi_pad, fw_pad = _fine_call(query[:, :KR], kg, cand_pad, KR)
    fine_global_idx = fgi_pad[:, :_FINE_K]
    fine_weights = fw_pad[:, :_FINE_K]

    vg = _sc_gather(V_all, fine_global_idx.reshape(-1), 64)
    vg = vg.reshape(n_rows, _FINE_K * D)
    output = _wsum_call(vg, fw_pad, _FINE_K, D)

    return (output.reshape(B, S, D),
            candidate_idx.reshape(B, S, _COARSE_K),
            coarse_scores.reshape(B, S, _COARSE_K),
            fine_global_idx.reshape(B, S, _FINE_K),
            fine_weights.reshape(B, S, _FINE_K))


# speedup vs baseline: 1.0656x; 1.0656x over previous
import functools
import math

import jax
import jax.numpy as jnp
from jax import lax
from jax.experimental import pallas as pl
from jax.experimental.pallas import tpu as pltpu
from jax.experimental.pallas import tpu_sc as plsc

_COARSE_K = 20
_FINE_K = 10
_SEG = 128
_BLK_R = 1024
_BLK_N = 2048
_BIG = 2 ** 30


def _coarse_body(x_ref, w_ref, wenc_ref, logits_ref, sel_ref, query_ref,
                 m_ref, carry_ref, *, nk, ck, g_real):
    i = pl.program_id(0)
    j = pl.program_id(1)
    nj = pl.num_programs(1)
    blk_r, blk_n = logits_ref.shape
    gper = blk_n // _SEG
    per_chunk = 128 // gper

    acc = jnp.dot(x_ref[...], w_ref[...], preferred_element_type=jnp.float32)
    cols = j * blk_n + lax.broadcasted_iota(jnp.int32, (blk_r, blk_n), 1)
    acc = jnp.where(cols < nk, acc, -jnp.inf)
    logits_ref[...] = acc

    @pl.when(j == 0)
    def _():
        query_ref[...] = jnp.dot(x_ref[...], wenc_ref[...],
                                 preferred_element_type=jnp.float32)

    pieces = [jnp.max(acc[:, g * _SEG:(g + 1) * _SEG], axis=1, keepdims=True)
              for g in range(gper)]
    tile_m = jnp.concatenate(pieces, axis=1)
    q = j % per_chunk

    @pl.when(q == 0)
    def _():
        carry_ref[...] = jnp.full(carry_ref.shape, -jnp.inf, jnp.float32)

    for qq in range(per_chunk):
        @pl.when(q == qq)
        def _(qq=qq):
            carry_ref[:, qq * gper:(qq + 1) * gper] = tile_m

    @pl.when((q == per_chunk - 1) | (j == nj - 1))
    def _():
        m_ref[:, pl.ds((j // per_chunk) * 128, 128)] = carry_ref[...]

    @pl.when(j == nj - 1)
    def _():
        g_tot = m_ref.shape[1]
        gi0 = lax.broadcasted_iota(jnp.int32, (blk_r, g_tot), 1)
        giota = gi0
        picks = []
        Mw = jnp.where(gi0 < g_real, m_ref[...], -jnp.inf)
        for _ in range(ck):
            m = jnp.max(Mw, axis=1, keepdims=True)
            pick = jnp.min(jnp.where(Mw == m, giota, _BIG),
                           axis=1, keepdims=True)
            picks.append(pick)
            Mw = jnp.where(giota == pick, -jnp.inf, Mw)
        gids = jnp.concatenate(picks, axis=1)
        outs = []
        for _ in range(ck):
            mn = jnp.min(gids, axis=1, keepdims=True)
            outs.append(mn)
            gids = jnp.where(gids == mn, _BIG, gids)
        gsorted = jnp.concatenate(outs, axis=1)
        rows = i * blk_r + lax.broadcasted_iota(jnp.int32, (blk_r, ck), 0)
        flat = rows * g_real + gsorted
        pad = jnp.zeros((blk_r, sel_ref.shape[1] - ck), jnp.int32)
        sel_ref[...] = jnp.concatenate([flat, pad], axis=1)


def _coarse_call(x2, wb, wenc, nk):
    n_rows, d = x2.shape
    blk_r = min(_BLK_R, n_rows)
    nj = (nk + _BLK_N - 1) // _BLK_N
    n_pad = nj * _BLK_N
    g_tot = n_pad // _SEG
    grid = (n_rows // blk_r, nj)
    g_pad = ((g_tot + 127) // 128) * 128
    body = functools.partial(_coarse_body, nk=nk, ck=_COARSE_K, g_real=g_tot)
    return pl.pallas_call(
        body,
        grid=grid,
        in_specs=[
            pl.BlockSpec((blk_r, d), lambda i, j: (i, 0)),
            pl.BlockSpec((d, _BLK_N), lambda i, j: (0, j)),
            pl.BlockSpec((d, 128), lambda i, j: (0, 0)),
        ],
        out_specs=[
            pl.BlockSpec((blk_r, _BLK_N), lambda i, j: (i, j)),
            pl.BlockSpec((blk_r, 128), lambda i, j: (i, 0)),
            pl.BlockSpec((blk_r, 128), lambda i, j: (i, 0)),
        ],
        out_shape=[
            jax.ShapeDtypeStruct((n_rows, n_pad), jnp.float32),
            jax.ShapeDtypeStruct((n_rows, 128), jnp.int32),
            jax.ShapeDtypeStruct((n_rows, 128), jnp.float32),
        ],
        scratch_shapes=[pltpu.VMEM((blk_r, g_pad), jnp.float32),
                        pltpu.VMEM((blk_r, 128), jnp.float32)],
        compiler_params=pltpu.CompilerParams(
            dimension_semantics=("parallel", "arbitrary")),
    )(x2, wb, wenc)


_SC_NC = 2
_SC_NS = 16


def _sc_gather(table, idx, chunk):
    n = idx.shape[0]
    drow = table.shape[1]
    nw = _SC_NC * _SC_NS
    b_per_w = n // nw
    assert n % nw == 0 and b_per_w % chunk == 0
    nch = b_per_w // chunk
    mesh = plsc.VectorSubcoreMesh(core_axis_name="c", subcore_axis_name="s")

    @functools.partial(
        pl.kernel,
        out_type=jax.ShapeDtypeStruct((n, drow), table.dtype),
        mesh=mesh,
        scratch_types=[
            pltpu.VMEM((chunk,), jnp.int32),
            pltpu.VMEM((chunk, drow), table.dtype),
            pltpu.SemaphoreType.DMA,
        ],
    )
    def k(table_hbm, idx_hbm, out_hbm, idx_v, rows_v, sem):
        wid = lax.axis_index("s") * _SC_NC + lax.axis_index("c")
        base = wid * b_per_w

        def body(c, carry):
            off = base + c * chunk
            pltpu.sync_copy(idx_hbm.at[pl.ds(off, chunk)], idx_v)
            pltpu.async_copy(table_hbm.at[idx_v], rows_v, sem).wait()
            pltpu.sync_copy(rows_v, out_hbm.at[pl.ds(off, chunk)])
            return carry

        lax.fori_loop(0, nch, body, 0)

    return k(table, idx)


_BLK_R2 = 256


def _refine_body(vals_ref, sel_ref, cand_ref, score_ref, *, ck, g_real):
    blk_r, width = vals_ref.shape
    vals = vals_ref[...]
    piota = lax.broadcasted_iota(jnp.int32, (blk_r, width), 1)
    gid = sel_ref[:, :ck] % g_real
    kiota = lax.broadcasted_iota(jnp.int32, (blk_r, ck), 1)
    mcols, gcols = [], []
    for _ in range(ck):
        m = jnp.max(vals, axis=1, keepdims=True)
        p = jnp.min(jnp.where(vals == m, piota, _BIG), axis=1, keepdims=True)
        vals = jnp.where(piota == p, -jnp.inf, vals)
        gid_p = jnp.min(jnp.where(kiota == p // _SEG, gid, _BIG),
                        axis=1, keepdims=True)
        g = gid_p * _SEG + p % _SEG
        mcols.append(m)
        gcols.append(g)
    pad_i = jnp.zeros((blk_r, cand_ref.shape[1] - ck), jnp.int32)
    pad_f = jnp.zeros((blk_r, score_ref.shape[1] - ck), jnp.float32)
    cand_ref[...] = jnp.concatenate(gcols + [pad_i], axis=1)
    score_ref[...] = jnp.concatenate(mcols + [pad_f], axis=1)


def _refine_call(vals, sel, g_real):
    n_rows, width = vals.shape
    blk_r = min(_BLK_R2, n_rows)
    grid = (n_rows // blk_r,)
    body = functools.partial(_refine_body, ck=_COARSE_K, g_real=g_real)
    return pl.pallas_call(
        body,
        grid=grid,
        in_specs=[
            pl.BlockSpec((blk_r, width), lambda i: (i, 0)),
            pl.BlockSpec((blk_r, 128), lambda i: (i, 0)),
        ],
        out_specs=[
            pl.BlockSpec((blk_r, 128), lambda i: (i, 0)),
            pl.BlockSpec((blk_r, 128), lambda i: (i, 0)),
        ],
        out_shape=[
            jax.ShapeDtypeStruct((n_rows, 128), jnp.int32),
            jax.ShapeDtypeStruct((n_rows, 128), jnp.float32),
        ],
        compiler_params=pltpu.CompilerParams(
            dimension_semantics=("parallel",)),
    )(vals, sel)


def _fine_body(q_ref, kg_ref, cand_ref, fgi_ref, fw_ref, *, ck, fk, kr):
    blk_r = q_ref.shape[0]
    qf = q_ref[...].astype(jnp.bfloat16).astype(jnp.float32)
    scols = []
    for c in range(ck):
        kc = kg_ref[:, c * kr:(c + 1) * kr].astype(jnp.bfloat16).astype(jnp.float32)
        s = jnp.sum(qf * kc, axis=1, keepdims=True) / math.sqrt(kr)
        scols.append(s)
    neg = jnp.full((blk_r, 128 - ck), -jnp.inf, jnp.float32)
    fs = jnp.concatenate(scols + [neg], axis=1)
    iota = lax.broadcasted_iota(jnp.int32, (blk_r, 128), 1)
    cand = cand_ref[...]
    tcols, icols = [], []
    for _ in range(fk):
        m = jnp.max(fs, axis=1, keepdims=True)
        p = jnp.min(jnp.where(fs == m, iota, _BIG), axis=1, keepdims=True)
        g = jnp.min(jnp.where(iota == p, cand, _BIG), axis=1, keepdims=True)
        tcols.append(m)
        icols.append(g)
        fs = jnp.where(iota == p, -jnp.inf, fs)
    fts = jnp.concatenate(tcols, axis=1)
    mx = fts[:, 0:1]
    e = jnp.exp(fts - mx)
    w = e / jnp.sum(e, axis=1, keepdims=True)
    pad_i = jnp.zeros((blk_r, fgi_ref.shape[1] - fk), jnp.int32)
    pad_f = jnp.zeros((blk_r, fw_ref.shape[1] - fk), jnp.float32)
    fgi_ref[...] = jnp.concatenate(icols + [pad_i], axis=1)
    fw_ref[...] = jnp.concatenate([w, pad_f], axis=1)


def _fine_call(query, kg, cand, kr):
    n_rows = query.shape[0]
    blk_r = min(_BLK_R2, n_rows)
    grid = (n_rows // blk_r,)
    width = kg.shape[1]
    body = functools.partial(_fine_body, ck=_COARSE_K, fk=_FINE_K, kr=kr)
    return pl.pallas_call(
        body,
        grid=grid,
        in_specs=[
            pl.BlockSpec((blk_r, query.shape[1]), lambda i: (i, 0)),
            pl.BlockSpec((blk_r, width), lambda i: (i, 0)),
            pl.BlockSpec((blk_r, 128), lambda i: (i, 0)),
        ],
        out_specs=[
            pl.BlockSpec((blk_r, 128), lambda i: (i, 0)),
            pl.BlockSpec((blk_r, 128), lambda i: (i, 0)),
        ],
        out_shape=[
            jax.ShapeDtypeStruct((n_rows, 128), jnp.int32),
            jax.ShapeDtypeStruct((n_rows, 128), jnp.float32),
        ],
        compiler_params=pltpu.CompilerParams(
            dimension_semantics=("parallel",)),
    )(query, kg, cand)


def _wsum_body(vg_ref, fw_ref, out_ref, *, fk, d):
    acc = vg_ref[:, 0:d] * fw_ref[:, 0:1]
    for j in range(1, fk):
        acc = acc + vg_ref[:, j * d:(j + 1) * d] * fw_ref[:, j:j + 1]
    out_ref[...] = acc


def _wsum_call(vg, fw, fk, d):
    n_rows = vg.shape[0]
    blk_r = min(_BLK_R2, n_rows)
    grid = (n_rows // blk_r,)
    body = functools.partial(_wsum_body, fk=fk, d=d)
    return pl.pallas_call(
        body,
        grid=grid,
        in_specs=[
            pl.BlockSpec((blk_r, vg.shape[1]), lambda i: (i, 0)),
            pl.BlockSpec((blk_r, 128), lambda i: (i, 0)),
        ],
        out_specs=pl.BlockSpec((blk_r, d), lambda i: (i, 0)),
        out_shape=jax.ShapeDtypeStruct((n_rows, d), jnp.float32),
        compiler_params=pltpu.CompilerParams(
            dimension_semantics=("parallel",)),
    )(vg, fw)


def kernel(x, W_router, W_enc, K_all, V_all):
    B, S, D = x.shape
    NK = W_router.shape[1]
    KR = K_all.shape[1]
    n_rows = B * S

    x2 = x.reshape(n_rows, D).astype(jnp.bfloat16)
    wb = W_router.astype(jnp.bfloat16)
    wenc = W_enc.astype(jnp.bfloat16)

    logits, sel, query = _coarse_call(x2, wb, wenc, NK)
    g_tot = logits.shape[1] // _SEG

    segs = logits.reshape(n_rows * g_tot, _SEG)
    selc = sel[:, :_COARSE_K].reshape(-1)
    gath = _sc_gather(segs, selc, 128)
    vals = gath.reshape(n_rows, _COARSE_K * _SEG)

    cand_pad, score_pad = _refine_call(vals, sel, g_tot)
    candidate_idx = cand_pad[:, :_COARSE_K]
    coarse_scores = score_pad[:, :_COARSE_K]

    kg = _sc_gather(K_all, candidate_idx.reshape(-1), 128)
    kg = kg.reshape(n_rows, _COARSE_K * KR)
    fgi_pad, fw_pad = _fine_call(query[:, :KR], kg, cand_pad, KR)
    fine_global_idx = fgi_pad[:, :_FINE_K]
    fine_weights = fw_pad[:, :_FINE_K]

    vg = _sc_gather(V_all, fine_global_idx.reshape(-1), 64)
    vg = vg.reshape(n_rows, _FINE_K * D)
    output = _wsum_call(vg, fw_pad, _FINE_K, D)

    return (output.reshape(B, S, D),
            candidate_idx.reshape(B, S, _COARSE_K),
            coarse_scores.reshape(B, S, _COARSE_K),
            fine_global_idx.reshape(B, S, _FINE_K),
            fine_weights.reshape(B, S, _FINE_K))
